# baseline (speedup 1.0000x reference)
import functools

import jax
import jax.numpy as jnp
from jax.experimental import pallas as pl
from jax.experimental.layout import Format, Layout, with_layout_constraint

_LOG2E = 1.4426950408889634


def _gat_kernel(xp_ref, adj_ref, asrc_ref, adst_ref, bias_ref, out_ref):
    N = adj_ref.shape[2]
    H = xp_ref.shape[3]

    xp = xp_ref[0, 0]

    s_src = jax.lax.dot_general(
        xp, asrc_ref[...], (((1,), (1,)), ((), ())),
        preferred_element_type=jnp.float32) * _LOG2E
    s_dst = jax.lax.dot_general(
        adst_ref[...], xp, (((1,), (1,)), ((), ())),
        preferred_element_type=jnp.float32) * _LOG2E

    s_max = jnp.max(s_src)
    mrow = s_max + s_dst
    mrow = jnp.maximum(mrow, 0.2 * mrow)

    logits = s_src + s_dst
    logits = jnp.maximum(logits, 0.2 * logits)

    row = jax.lax.broadcasted_iota(jnp.int32, (N, N), 0)
    col = jax.lax.broadcasted_iota(jnp.int32, (N, N), 1)
    mask = (adj_ref[0, 0] != 0) | (row == col)
    e = jnp.where(mask, jnp.exp2(logits - mrow), 0.0)

    ones = jnp.ones((N, 1), dtype=jnp.float32)
    xp_aug = jnp.concatenate([xp, ones], axis=1)
    num_aug = jax.lax.dot_general(
        e.astype(jnp.bfloat16), xp_aug.astype(jnp.bfloat16),
        (((0,), (0,)), ((), ())),
        preferred_element_type=jnp.float32)

    denom = num_aug[:, H:H + 1] + 1e-16
    out_ref[0, 0] = num_aug[:, :H] / denom + bias_ref[...]


def _gat(x, adj_matrix, W, a_src, a_dst, bias, fmt=None):
    B, S, N, F = x.shape
    H = W.shape[1]

    xp = jnp.einsum("bsnf,fh->bsnh", x, W,
                    preferred_element_type=jnp.float32)
    if fmt is not None:
        xp = with_layout_constraint(xp, fmt.layout)
    a_src2 = a_src.reshape(1, H)
    a_dst2 = a_dst.reshape(1, H)
    bias2 = bias.reshape(1, H)

    return pl.pallas_call(
        _gat_kernel,
        grid=(B, S),
        in_specs=[
            pl.BlockSpec((1, 1, N, H), lambda b, s: (b, s, 0, 0)),
            pl.BlockSpec((1, 1, N, N), lambda b, s: (b, s, 0, 0)),
            pl.BlockSpec((1, H), lambda b, s: (0, 0)),
            pl.BlockSpec((1, H), lambda b, s: (0, 0)),
            pl.BlockSpec((1, H), lambda b, s: (0, 0)),
        ],
        out_specs=pl.BlockSpec((1, 1, N, H), lambda b, s: (b, s, 0, 0)),
        out_shape=jax.ShapeDtypeStruct((B, S, N, H), jnp.float32),
    )(xp, adj_matrix, a_src2, a_dst2, bias2)


_jitted = None


def kernel(x, adj_matrix, W, a_src, a_dst, bias):
    global _jitted
    if _jitted is None:
        try:
            fmt = Format(Layout(major_to_minor=(3, 2, 1, 0), tiling=((8, 128),)),
                         jax.sharding.SingleDeviceSharding(jax.devices()[0]))
            _jitted = jax.jit(functools.partial(_gat, fmt=fmt),
                              out_shardings=fmt)
        except Exception:
            _jitted = jax.jit(_gat)
    return _jitted(x, adj_matrix, W, a_src, a_dst, bias)


# speedup vs baseline: 1.0311x; 1.0311x over previous
import jax
import jax.numpy as jnp
from jax.experimental import pallas as pl

_LOG2E = 1.4426950408889634


def _gat_kernel(xp_ref, adj_ref, asrc_ref, adst_ref, bias_ref, out_ref):
    N = adj_ref.shape[2]
    H = xp_ref.shape[3]

    xp = xp_ref[0, 0]
    xp32 = xp.astype(jnp.float32)

    s_src = jax.lax.dot_general(
        xp32, asrc_ref[...], (((1,), (1,)), ((), ())),
        preferred_element_type=jnp.float32) * _LOG2E
    s_dst = jax.lax.dot_general(
        adst_ref[...], xp32, (((1,), (1,)), ((), ())),
        preferred_element_type=jnp.float32) * _LOG2E

    s_max = jnp.max(s_src)
    mrow = s_max + s_dst
    mrow = jnp.maximum(mrow, 0.2 * mrow)

    logits = s_src + s_dst
    logits = jnp.maximum(logits, 0.2 * logits)

    row = jax.lax.broadcasted_iota(jnp.int32, (N, N), 0)
    col = jax.lax.broadcasted_iota(jnp.int32, (N, N), 1)
    mask = (adj_ref[0, 0] != 0) | (row == col)
    e = jnp.where(mask, jnp.exp2(logits - mrow), 0.0)

    ones = jnp.ones((N, 1), dtype=jnp.bfloat16)
    xp_aug = jnp.concatenate([xp, ones], axis=1)
    num_aug = jax.lax.dot_general(
        e.astype(jnp.bfloat16), xp_aug,
        (((0,), (0,)), ((), ())),
        preferred_element_type=jnp.float32)

    denom = num_aug[:, H:H + 1] + 1e-16
    out_ref[0, 0] = num_aug[:, :H] / denom + bias_ref[...]


@jax.jit
def kernel(x, adj_matrix, W, a_src, a_dst, bias):
    B, S, N, F = x.shape
    H = W.shape[1]

    xp = jnp.einsum("bsnf,fh->bsnh", x, W,
                    preferred_element_type=jnp.float32).astype(jnp.bfloat16)
    a_src2 = a_src.reshape(1, H)
    a_dst2 = a_dst.reshape(1, H)
    bias2 = bias.reshape(1, H)

    return pl.pallas_call(
        _gat_kernel,
        grid=(B, S),
        in_specs=[
            pl.BlockSpec((1, 1, N, H), lambda b, s: (b, s, 0, 0)),
            pl.BlockSpec((1, 1, N, N), lambda b, s: (b, s, 0, 0)),
            pl.BlockSpec((1, H), lambda b, s: (0, 0)),
            pl.BlockSpec((1, H), lambda b, s: (0, 0)),
            pl.BlockSpec((1, H), lambda b, s: (0, 0)),
        ],
        out_specs=pl.BlockSpec((1, 1, N, H), lambda b, s: (b, s, 0, 0)),
        out_shape=jax.ShapeDtypeStruct((B, S, N, H), jnp.float32),
    )(xp, adj_matrix, a_src2, a_dst2, bias2)


# speedup vs baseline: 1.0849x; 1.0522x over previous
import jax
import jax.numpy as jnp
from jax.experimental import pallas as pl

_LOG2E = 1.4426950408889634


def _gat_kernel(xp_ref, adj_ref, asrc_ref, adst_ref, bias_ref, out_ref):
    N = adj_ref.shape[2]
    H = xp_ref.shape[3]

    xp = xp_ref[0, 0]
    xp32 = xp.astype(jnp.float32)

    s_src = jax.lax.dot_general(
        xp32, asrc_ref[...], (((1,), (1,)), ((), ())),
        preferred_element_type=jnp.float32) * _LOG2E
    s_dst = jax.lax.dot_general(
        adst_ref[...], xp32, (((1,), (1,)), ((), ())),
        preferred_element_type=jnp.float32) * _LOG2E

    s_max = jnp.max(s_src)
    mrow = s_max + s_dst
    mrow = jnp.maximum(mrow, 0.2 * mrow)

    logits = s_src + s_dst
    logits = jnp.maximum(logits, 0.2 * logits)

    row = jax.lax.broadcasted_iota(jnp.int32, (N, N), 0)
    col = jax.lax.broadcasted_iota(jnp.int32, (N, N), 1)
    mask = (adj_ref[0, 0] != 0) | (row == col)
    e = jnp.where(mask, jnp.exp2(logits - mrow), 0.0)

    ones = jnp.ones((N, 1), dtype=jnp.bfloat16)
    xp_aug = jnp.concatenate([xp, ones], axis=1)
    num_aug = jax.lax.dot_general(
        e.astype(jnp.bfloat16), xp_aug,
        (((0,), (0,)), ((), ())),
        preferred_element_type=jnp.float32)

    denom = num_aug[:, H:H + 1] + 1e-16
    out = num_aug[:, :H] / denom + bias_ref[...]
    out_ref[0, 0] = out.astype(jnp.bfloat16)


@jax.jit
def kernel(x, adj_matrix, W, a_src, a_dst, bias):
    B, S, N, F = x.shape
    H = W.shape[1]

    xp = jnp.einsum("bsnf,fh->bsnh", x, W,
                    preferred_element_type=jnp.float32).astype(jnp.bfloat16)
    a_src2 = a_src.reshape(1, H)
    a_dst2 = a_dst.reshape(1, H)
    bias2 = bias.reshape(1, H)

    out = pl.pallas_call(
        _gat_kernel,
        grid=(B, S),
        in_specs=[
            pl.BlockSpec((1, 1, N, H), lambda b, s: (b, s, 0, 0)),
            pl.BlockSpec((1, 1, N, N), lambda b, s: (b, s, 0, 0)),
            pl.BlockSpec((1, H), lambda b, s: (0, 0)),
            pl.BlockSpec((1, H), lambda b, s: (0, 0)),
            pl.BlockSpec((1, H), lambda b, s: (0, 0)),
        ],
        out_specs=pl.BlockSpec((1, 1, N, H), lambda b, s: (b, s, 0, 0)),
        out_shape=jax.ShapeDtypeStruct((B, S, N, H), jnp.bfloat16),
    )(xp, adj_matrix, a_src2, a_dst2, bias2)
    return out.astype(jnp.float32)


# speedup vs baseline: 1.0865x; 1.0015x over previous
import jax
import jax.numpy as jnp
from jax.experimental import pallas as pl
from jax.experimental.layout import Layout, with_layout_constraint

_LOG2E = 1.4426950408889634


def _gat_kernel(xp_ref, adj_ref, asrc_ref, adst_ref, bias_ref, out_ref):
    N = adj_ref.shape[2]
    H = xp_ref.shape[3]

    xp = xp_ref[0, 0]
    xp32 = xp.astype(jnp.float32)

    s_src = jax.lax.dot_general(
        xp32, asrc_ref[...], (((1,), (1,)), ((), ())),
        preferred_element_type=jnp.float32) * _LOG2E
    s_dst = jax.lax.dot_general(
        adst_ref[...], xp32, (((1,), (1,)), ((), ())),
        preferred_element_type=jnp.float32) * _LOG2E

    s_max = jnp.max(s_src)
    mrow = s_max + s_dst
    mrow = jnp.maximum(mrow, 0.2 * mrow)

    logits = s_src + s_dst
    logits = jnp.maximum(logits, 0.2 * logits)

    row = jax.lax.broadcasted_iota(jnp.int32, (N, N), 0)
    col = jax.lax.broadcasted_iota(jnp.int32, (N, N), 1)
    mask = (adj_ref[0, 0] != 0) | (row == col)
    e = jnp.where(mask, jnp.exp2(logits - mrow), 0.0)

    ones = jnp.ones((N, 1), dtype=jnp.bfloat16)
    xp_aug = jnp.concatenate([xp, ones], axis=1)
    num_aug = jax.lax.dot_general(
        e.astype(jnp.bfloat16), xp_aug,
        (((0,), (0,)), ((), ())),
        preferred_element_type=jnp.float32)

    denom = num_aug[:, H:H + 1] + 1e-16
    out = num_aug[:, :H] / denom + bias_ref[...]
    out_ref[0, 0] = out.astype(jnp.bfloat16)


@jax.jit
def kernel(x, adj_matrix, W, a_src, a_dst, bias):
    B, S, N, F = x.shape
    H = W.shape[1]

    xp = jnp.einsum("bsnf,fh->bsnh", x, W,
                    preferred_element_type=jnp.float32).astype(jnp.bfloat16)
    xp = with_layout_constraint(
        xp, Layout(major_to_minor=(3, 2, 1, 0), tiling=((16, 128), (2, 1))))
    a_src2 = a_src.reshape(1, H)
    a_dst2 = a_dst.reshape(1, H)
    bias2 = bias.reshape(1, H)

    out = pl.pallas_call(
        _gat_kernel,
        grid=(B, S),
        in_specs=[
            pl.BlockSpec((1, 1, N, H), lambda b, s: (b, s, 0, 0)),
            pl.BlockSpec((1, 1, N, N), lambda b, s: (b, s, 0, 0)),
            pl.BlockSpec((1, H), lambda b, s: (0, 0)),
            pl.BlockSpec((1, H), lambda b, s: (0, 0)),
            pl.BlockSpec((1, H), lambda b, s: (0, 0)),
        ],
        out_specs=pl.BlockSpec((1, 1, N, H), lambda b, s: (b, s, 0, 0)),
        out_shape=jax.ShapeDtypeStruct((B, S, N, H), jnp.bfloat16),
    )(xp, adj_matrix, a_src2, a_dst2, bias2)
    return out.astype(jnp.float32)
